# baseline (speedup 1.0000x reference)
import functools

import jax
import jax.numpy as jnp
from jax import lax
from jax.experimental import pallas as pl
from jax.experimental.pallas import tpu as pltpu
from jax.experimental.pallas import tpu_sc as plsc

_MAX = 255
_D = 20
_NC, _NS, _L = 2, 16, 16
_NW = _NC * _NS
_RB = 16


@jax.jit
def _gather(table, tensor_i32):
    rows, feat = tensor_i32.shape
    rows_w = rows // _NW
    nblk = rows_w // _RB
    assert nblk % 2 == 0 and nblk >= 4
    mesh = plsc.VectorSubcoreMesh(core_axis_name="c", subcore_axis_name="s")

    @functools.partial(
        pl.kernel,
        out_type=jax.ShapeDtypeStruct((rows, feat * _D), jnp.float32),
        mesh=mesh,
        compiler_params=pltpu.CompilerParams(needs_layout_passes=False),
        scratch_types=[
            pltpu.VMEM(((_MAX + 1) * _D,), jnp.float32),
            pltpu.VMEM((_RB, feat), jnp.int32),
            pltpu.VMEM((_RB, feat), jnp.int32),
            pltpu.VMEM((_RB, feat * _D), jnp.float32),
            pltpu.VMEM((_RB, feat * _D), jnp.float32),
            pltpu.SemaphoreType.DMA,
            pltpu.SemaphoreType.DMA,
            pltpu.SemaphoreType.DMA,
            pltpu.SemaphoreType.DMA,
        ],
    )
    def k(table_hbm, t_hbm, out_hbm, table_v,
          idx0, idx1, outv0, outv1, isem0, isem1, osem0, osem1):
        wid = lax.axis_index("s") * _NC + lax.axis_index("c")
        pltpu.sync_copy(table_hbm, table_v)
        lanes = lax.iota(jnp.int32, _L)
        row0 = wid * rows_w

        idx_bufs = (idx0, idx1)
        out_bufs = (outv0, outv1)
        isems = (isem0, isem1)
        osems = (osem0, osem1)

        pltpu.async_copy(t_hbm.at[pl.ds(row0, _RB), :], idx0, isem0)
        pltpu.async_copy(t_hbm.at[pl.ds(row0 + _RB, _RB), :], idx1, isem1)

        def half(j, t):
            b = j * 2 + t
            rbase = row0 + b * _RB
            idx_v, out_v = idx_bufs[t], out_bufs[t]
            isem, osem = isems[t], osems[t]
            pltpu.make_async_copy(
                t_hbm.at[pl.ds(rbase, _RB), :], idx_v, isem).wait()

            @pl.when(j > 0)
            def _():
                pltpu.make_async_copy(
                    out_v, out_hbm.at[pl.ds(rbase, _RB), :], osem).wait()

            dvec = lax.iota(jnp.int32, _L)
            starts = list(range(0, feat - _L + 1, _L))
            if starts[-1] != feat - _L:
                starts.append(feat - _L)

            @plsc.parallel_loop(0, _RB * feat)
            def _cp(it):
                r = it // feat
                f = it - r * feat
                s = jnp.minimum((f // _L) * _L, feat - _L)
                jl = f - s
                ivv = idx_v[r, pl.ds(s, _L)]
                ivb = ivv[jnp.full((_L,), jl, jnp.int32)]
                ivb = jnp.minimum(jnp.maximum(ivb, 0), _MAX) * _D
                a0 = ivb + dvec
                v0 = plsc.load_gather(table_v, [a0])
                v1 = plsc.load_gather(table_v, [a0 + (_D - _L)])
                rv = jnp.full((_L,), r, jnp.int32)
                colv = dvec + f * _D
                plsc.store_scatter(out_v, [rv, colv], v0)
                plsc.store_scatter(out_v, [rv, colv + (_D - _L)], v1)

            @pl.when(b + 2 < nblk)
            def _():
                pltpu.async_copy(
                    t_hbm.at[pl.ds(rbase + 2 * _RB, _RB), :], idx_v, isem)

            pltpu.async_copy(out_v, out_hbm.at[pl.ds(rbase, _RB), :], osem)

        def blk2(j, carry):
            half(j, 0)
            half(j, 1)
            return carry

        lax.fori_loop(0, nblk // 2, blk2, 0)
        pltpu.make_async_copy(
            outv0,
            out_hbm.at[pl.ds(row0 + (nblk - 2) * _RB, _RB), :], osem0).wait()
        pltpu.make_async_copy(
            outv1,
            out_hbm.at[pl.ds(row0 + (nblk - 1) * _RB, _RB), :], osem1).wait()

    return k(table, tensor_i32)


def kernel(tensor, int_to_feat_matrix, extra_embeddings):
    orig_shape = tensor.shape
    t2 = tensor.reshape(-1, orig_shape[-1]).astype(jnp.int32)
    table = jnp.concatenate(
        [int_to_feat_matrix, extra_embeddings[:1]], axis=0).reshape(-1)
    out = _gather(table, t2)
    return out.reshape(*orig_shape[:-1], orig_shape[-1] * _D)


# speedup vs baseline: 1.5715x; 1.5715x over previous
import functools

import jax
import jax.numpy as jnp
from jax import lax
from jax.experimental import pallas as pl
from jax.experimental.pallas import tpu as pltpu
from jax.experimental.pallas import tpu_sc as plsc

_MAX = 255
_D = 20
_NC, _NS, _L = 2, 16, 16
_NW = _NC * _NS
_BL = 2560


@functools.partial(jax.jit, static_argnums=(2, 3))
def _gather(table, idx, n, bl):
    per_w = n // _NW
    nblk = per_w // bl
    assert nblk % 2 == 0 and nblk >= 4
    mesh = plsc.VectorSubcoreMesh(core_axis_name="c", subcore_axis_name="s")

    @functools.partial(
        pl.kernel,
        out_type=jax.ShapeDtypeStruct((n * _D,), jnp.float32),
        mesh=mesh,
        compiler_params=pltpu.CompilerParams(needs_layout_passes=False),
        scratch_types=[
            pltpu.VMEM(((_MAX + 1) * _D,), jnp.float32),
            pltpu.VMEM((bl,), jnp.int32),
            pltpu.VMEM((bl,), jnp.int32),
            pltpu.VMEM((bl * _D,), jnp.float32),
            pltpu.VMEM((bl * _D,), jnp.float32),
            pltpu.SemaphoreType.DMA,
            pltpu.SemaphoreType.DMA,
            pltpu.SemaphoreType.DMA,
            pltpu.SemaphoreType.DMA,
        ],
    )
    def k(table_hbm, idx_hbm, out_hbm, table_v,
          idx0, idx1, outv0, outv1, isem0, isem1, osem0, osem1):
        wid = lax.axis_index("s") * _NC + lax.axis_index("c")
        pltpu.sync_copy(table_hbm, table_v)
        lane = lax.iota(jnp.int32, _L) * _D
        base0 = wid * per_w
        tl = (_MAX + 1) * _D - (_D - 1)
        sl = (_L - 1) * _D + 1

        idx_bufs = (idx0, idx1)
        out_bufs = (outv0, outv1)
        isems = (isem0, isem1)
        osems = (osem0, osem1)

        pltpu.async_copy(idx_hbm.at[pl.ds(base0, bl)], idx0, isem0)
        pltpu.async_copy(idx_hbm.at[pl.ds(base0 + bl, bl)], idx1, isem1)

        def half(j, t):
            b = j * 2 + t
            base = base0 + b * bl
            idx_v, out_v = idx_bufs[t], out_bufs[t]
            isem, osem = isems[t], osems[t]
            pltpu.make_async_copy(idx_hbm.at[pl.ds(base, bl)], idx_v, isem).wait()
            @pl.when(j > 0)
            def _():
                pltpu.make_async_copy(
                    out_v, out_hbm.at[pl.ds(base * _D, bl * _D)], osem).wait()

            @plsc.parallel_loop(0, bl // _L)
            def _grp(g):
                iv = idx_v[pl.ds(g * _L, _L)]
                iv = jnp.minimum(jnp.maximum(iv, 0), _MAX) * _D
                obase = lane + g * (_L * _D)
                vals = [plsc.load_gather(table_v, [iv + d]) for d in range(_D)]
                for d in range(_D):
                    plsc.store_scatter(out_v, [obase + d], vals[d])
            pltpu.async_copy(out_v, out_hbm.at[pl.ds(base * _D, bl * _D)], osem)
            @pl.when(b + 2 < nblk)
            def _():
                pltpu.async_copy(
                    idx_hbm.at[pl.ds(base + 2 * bl, bl)], idx_v, isem)

        def blk2(j, carry):
            half(j, 0)
            half(j, 1)
            return carry

        lax.fori_loop(0, nblk // 2, blk2, 0)
        last0 = base0 + (nblk - 2) * bl
        last1 = base0 + (nblk - 1) * bl
        pltpu.make_async_copy(
            outv0, out_hbm.at[pl.ds(last0 * _D, bl * _D)], osem0).wait()
        pltpu.make_async_copy(
            outv1, out_hbm.at[pl.ds(last1 * _D, bl * _D)], osem1).wait()

    return k(table, idx)


def kernel(tensor, int_to_feat_matrix, extra_embeddings):
    orig_shape = tensor.shape
    idx = tensor.reshape(-1).astype(jnp.int32)
    n = idx.shape[0]
    table = jnp.concatenate(
        [int_to_feat_matrix, extra_embeddings[:1]], axis=0).reshape(-1)
    out = _gather(table, idx, n, _BL)
    return out.reshape(*orig_shape[:-1], orig_shape[-1] * _D)


# speedup vs baseline: 1.6212x; 1.0316x over previous
import functools

import jax
import jax.numpy as jnp
from jax import lax
from jax.experimental import pallas as pl
from jax.experimental.pallas import tpu as pltpu
from jax.experimental.pallas import tpu_sc as plsc

_MAX = 255
_D = 20
_NC, _NS, _L = 2, 16, 16
_NW = _NC * _NS
_BL = 2560


@functools.partial(jax.jit, static_argnums=(2, 3))
def _gather(table, idx, n, bl):
    per_w = n // _NW
    nblk = per_w // bl
    assert nblk % 2 == 0 and nblk >= 4
    mesh = plsc.VectorSubcoreMesh(core_axis_name="c", subcore_axis_name="s")

    @functools.partial(
        pl.kernel,
        out_type=jax.ShapeDtypeStruct((n * _D,), jnp.float32),
        mesh=mesh,
        compiler_params=pltpu.CompilerParams(needs_layout_passes=False),
        scratch_types=[
            pltpu.VMEM(((_MAX + 1) * (_D + 1),), jnp.float32),
            pltpu.VMEM((bl,), jnp.int32),
            pltpu.VMEM((bl,), jnp.int32),
            pltpu.VMEM((bl * _D,), jnp.float32),
            pltpu.VMEM((bl * _D,), jnp.float32),
            pltpu.SemaphoreType.DMA,
            pltpu.SemaphoreType.DMA,
            pltpu.SemaphoreType.DMA,
            pltpu.SemaphoreType.DMA,
        ],
    )
    def k(table_hbm, idx_hbm, out_hbm, table_v,
          idx0, idx1, outv0, outv1, isem0, isem1, osem0, osem1):
        wid = lax.axis_index("s") * _NC + lax.axis_index("c")
        pltpu.sync_copy(table_hbm, table_v)
        lane = lax.iota(jnp.int32, _L) * _D
        base0 = wid * per_w
        tl = (_MAX + 1) * _D - (_D - 1)
        sl = (_L - 1) * _D + 1

        idx_bufs = (idx0, idx1)
        out_bufs = (outv0, outv1)
        isems = (isem0, isem1)
        osems = (osem0, osem1)

        pltpu.async_copy(idx_hbm.at[pl.ds(base0, bl)], idx0, isem0)
        pltpu.async_copy(idx_hbm.at[pl.ds(base0 + bl, bl)], idx1, isem1)

        def half(j, t):
            b = j * 2 + t
            base = base0 + b * bl
            idx_v, out_v = idx_bufs[t], out_bufs[t]
            isem, osem = isems[t], osems[t]
            pltpu.make_async_copy(idx_hbm.at[pl.ds(base, bl)], idx_v, isem).wait()
            @pl.when(j > 0)
            def _():
                pltpu.make_async_copy(
                    out_v, out_hbm.at[pl.ds(base * _D, bl * _D)], osem).wait()

            @plsc.parallel_loop(0, bl // _L)
            def _grp(g):
                iv = idx_v[pl.ds(g * _L, _L)]
                iv = jnp.minimum(jnp.maximum(iv, 0), _MAX) * (_D + 1)
                obase = lane + g * (_L * _D)
                vals = [plsc.load_gather(table_v, [iv + d]) for d in range(_D)]
                for d in range(_D):
                    plsc.store_scatter(out_v, [obase + d], vals[d])
            pltpu.async_copy(out_v, out_hbm.at[pl.ds(base * _D, bl * _D)], osem)
            @pl.when(b + 2 < nblk)
            def _():
                pltpu.async_copy(
                    idx_hbm.at[pl.ds(base + 2 * bl, bl)], idx_v, isem)

        def blk2(j, carry):
            half(j, 0)
            half(j, 1)
            return carry

        lax.fori_loop(0, nblk // 2, blk2, 0)
        last0 = base0 + (nblk - 2) * bl
        last1 = base0 + (nblk - 1) * bl
        pltpu.make_async_copy(
            outv0, out_hbm.at[pl.ds(last0 * _D, bl * _D)], osem0).wait()
        pltpu.make_async_copy(
            outv1, out_hbm.at[pl.ds(last1 * _D, bl * _D)], osem1).wait()

    return k(table, idx)


def kernel(tensor, int_to_feat_matrix, extra_embeddings):
    orig_shape = tensor.shape
    idx = tensor.reshape(-1).astype(jnp.int32)
    n = idx.shape[0]
    table = jnp.concatenate(
        [int_to_feat_matrix, extra_embeddings[:1]], axis=0)
    table = jnp.pad(table, ((0, 0), (0, 1))).reshape(-1)
    out = _gather(table, idx, n, _BL)
    return out.reshape(*orig_shape[:-1], orig_shape[-1] * _D)


# speedup vs baseline: 1.8049x; 1.1133x over previous
import functools

import jax
import jax.numpy as jnp
from jax import lax
from jax.experimental import pallas as pl
from jax.experimental.pallas import tpu as pltpu
from jax.experimental.pallas import tpu_sc as plsc

_MAX = 255
_D = 20
_TS = _D + 1
_NC, _NS, _L = 2, 16, 16
_NW = _NC * _NS
_RB = 16


@functools.partial(jax.jit, static_argnums=(2, 3))
def _gather(table, idx, rows, feat):
    n = rows * feat
    per_w = rows // _NW
    nblk = per_w // _RB
    assert nblk % 2 == 0 and nblk >= 4
    bl = _RB * feat
    fd = feat * _D
    mesh = plsc.VectorSubcoreMesh(core_axis_name="c", subcore_axis_name="s")

    @functools.partial(
        pl.kernel,
        out_type=jax.ShapeDtypeStruct((rows, fd), jnp.float32),
        mesh=mesh,
        compiler_params=pltpu.CompilerParams(needs_layout_passes=False),
        scratch_types=[
            pltpu.VMEM(((_MAX + 1) * _TS,), jnp.float32),
            pltpu.VMEM((bl,), jnp.int32),
            pltpu.VMEM((bl,), jnp.int32),
            pltpu.VMEM((bl * _D,), jnp.float32),
            pltpu.VMEM((_RB, fd), jnp.float32),
            pltpu.VMEM((_RB, fd), jnp.float32),
            pltpu.SemaphoreType.DMA,
            pltpu.SemaphoreType.DMA,
            pltpu.SemaphoreType.DMA,
            pltpu.SemaphoreType.DMA,
        ],
    )
    def k(table_hbm, idx_hbm, out_hbm, table_v,
          idx0, idx1, out_f, outv0, outv1, isem0, isem1, osem0, osem1):
        wid = lax.axis_index("s") * _NC + lax.axis_index("c")
        pltpu.sync_copy(table_hbm, table_v)
        lane = lax.iota(jnp.int32, _L) * _D
        row0 = wid * per_w

        idx_bufs = (idx0, idx1)
        out_bufs = (outv0, outv1)
        isems = (isem0, isem1)
        osems = (osem0, osem1)

        pltpu.async_copy(idx_hbm.at[pl.ds(row0 * feat, bl)], idx0, isem0)
        pltpu.async_copy(
            idx_hbm.at[pl.ds((row0 + _RB) * feat, bl)], idx1, isem1)

        def half(j, t):
            b = j * 2 + t
            rbase = row0 + b * _RB
            idx_v, out_v = idx_bufs[t], out_bufs[t]
            isem, osem = isems[t], osems[t]
            pltpu.make_async_copy(
                idx_hbm.at[pl.ds(rbase * feat, bl)], idx_v, isem).wait()

            @pl.when(j > 0)
            def _():
                pltpu.make_async_copy(
                    out_v, out_hbm.at[pl.ds(rbase, _RB), :], osem).wait()

            @plsc.parallel_loop(0, bl // _L)
            def _grp(g):
                iv = idx_v[pl.ds(g * _L, _L)]
                iv = jnp.minimum(jnp.maximum(iv, 0), _MAX) * _TS
                obase = lane + g * (_L * _D)
                vals = [plsc.load_gather(table_v, [iv + d]) for d in range(_D)]
                for d in range(_D):
                    plsc.store_scatter(out_f, [obase + d], vals[d])

            @plsc.parallel_loop(0, _RB)
            def _rearr(r):
                for t in range((fd + 127) // 128):
                    nk = min(8, (fd - t * 128 + 15) // 16)
                    for kk in range(nk):
                        c = t * 128 + kk * 16
                        out_v[r, pl.ds(c, _L)] = out_f[pl.ds(r * fd + c, _L)]

            pltpu.async_copy(out_v, out_hbm.at[pl.ds(rbase, _RB), :], osem)

            @pl.when(b + 2 < nblk)
            def _():
                pltpu.async_copy(
                    idx_hbm.at[pl.ds((rbase + 2 * _RB) * feat, bl)],
                    idx_v, isem)

        def blk2(j, carry):
            half(j, 0)
            half(j, 1)
            return carry

        lax.fori_loop(0, nblk // 2, blk2, 0)
        pltpu.make_async_copy(
            outv0,
            out_hbm.at[pl.ds(row0 + (nblk - 2) * _RB, _RB), :], osem0).wait()
        pltpu.make_async_copy(
            outv1,
            out_hbm.at[pl.ds(row0 + (nblk - 1) * _RB, _RB), :], osem1).wait()

    return k(table, idx)


def kernel(tensor, int_to_feat_matrix, extra_embeddings):
    orig_shape = tensor.shape
    feat = orig_shape[-1]
    idx = tensor.reshape(-1).astype(jnp.int32)
    rows = idx.shape[0] // feat
    table = jnp.concatenate(
        [int_to_feat_matrix, extra_embeddings[:1]], axis=0)
    table = jnp.pad(table, ((0, 0), (0, 1))).reshape(-1)
    out = _gather(table, idx, rows, feat)
    return out.reshape(*orig_shape[:-1], feat * _D)
